# Initial kernel scaffold; baseline (speedup 1.0000x reference)
import jax
import jax.numpy as jnp
from jax.experimental import pallas as pl


def kernel(x, Wg, bg, We, be):
    raise NotImplementedError("write your pallas kernel here")


# speedup vs baseline: 3.8846x; 3.8846x over previous
import functools

import jax
import jax.numpy as jnp
from jax.experimental import pallas as pl
from jax.experimental.pallas import tpu as pltpu

_B = 2048
_D = 1024
_E = 8
_KEEP = 6
_EXPERT_ID = 0
_BM = 512


def _body(x_ref, Wg_ref, bg_ref, We_ref, be_ref, out_ref, acc_ref, en_ref, g_ref):
    e = pl.program_id(1)
    xb = x_ref[...]

    W = We_ref[0]
    o = jax.lax.dot_general(xb, W, (((1,), (1,)), ((), ())),
                            preferred_element_type=jnp.float32) + be_ref[0]
    acc_ref[e] = o

    @pl.when(e == 0)
    def _gate():
        logits = jax.lax.dot_general(Wg_ref[...], xb, (((0,), (1,)), ((), ())),
                                     preferred_element_type=jnp.float32)
        logits = logits + bg_ref[...]
        m = jnp.max(logits, axis=0, keepdims=True)
        ex = jnp.exp(logits - m)
        g_ref[...] = ex / jnp.sum(ex, axis=0, keepdims=True)
        ones = jnp.ones_like(xb)
        en_ref[0] = jnp.log(jnp.sum(jnp.exp(ones), axis=-1, keepdims=True).T)

    @pl.when(e > 0)
    def _energy():
        r = acc_ref[0]
        cos = (r * o) / (jnp.abs(r) * jnp.abs(o) + 1e-08)
        s = jnp.sum(jnp.exp(cos), axis=-1, keepdims=True)
        en_ref[e] = jnp.log(s).T

    @pl.when(e == _E - 1)
    def _combine():
        g = [g_ref[i:i + 1, :] for i in range(_E)]
        en = [en_ref[i, :, :] for i in range(_E)]

        rk = []
        for i in range(_E):
            acc = jnp.zeros_like(g[0], dtype=jnp.int32)
            for j in range(_E):
                if j == i:
                    continue
                if j < i:
                    acc = acc + (g[j] >= g[i]).astype(jnp.int32)
                else:
                    acc = acc + (g[j] > g[i]).astype(jnp.int32)
            rk.append(acc)

        EN, WV, IDX = [], [], []
        for k in range(_E):
            enk = jnp.zeros_like(en[0])
            wvk = jnp.zeros_like(g[0])
            idk = jnp.zeros_like(rk[0])
            for i in range(_E):
                hit = rk[i] == k
                enk = jnp.where(hit, en[i], enk)
                wvk = jnp.where(hit, g[i], wvk)
                idk = jnp.where(hit, i, idk)
            EN.append(enk)
            WV.append(wvk)
            IDX.append(idk)

        kept = []
        for k in range(_E):
            acc = jnp.zeros_like(rk[0])
            for j in range(_E):
                if j == k:
                    continue
                if j < k:
                    acc = acc + (EN[j] <= EN[k]).astype(jnp.int32)
                else:
                    acc = acc + (EN[j] < EN[k]).astype(jnp.int32)
            kept.append(acc < _KEEP)

        t = []
        for k in range(_E):
            tk = jnp.zeros_like(IDX[0])
            for p in range(_E):
                tk = jnp.where(IDX[k] == p, IDX[p], tk)
            t.append(tk)

        zero = jnp.zeros_like(g[0])
        out = None
        for i in range(_E):
            c = zero
            for k in range(_E):
                c = c + jnp.where(kept[k] & (t[k] == i), WV[k], zero)
            term = c.T * acc_ref[i]
            out = term if out is None else out + term
        out_ref[...] = out


@jax.jit
def kernel(x, Wg, bg, We, be):
    grid = (_B // _BM, _E)
    f = pl.pallas_call(
        _body,
        grid=grid,
        in_specs=[
            pl.BlockSpec((_BM, _D), lambda rb, e: (rb, 0)),
            pl.BlockSpec((_D, _E), lambda rb, e: (0, 0)),
            pl.BlockSpec((_E, 1), lambda rb, e: (0, 0)),
            pl.BlockSpec((1, _D, _D), lambda rb, e: (e, 0, 0)),
            pl.BlockSpec((1, 1, _D), lambda rb, e: (e, 0, 0)),
        ],
        out_specs=pl.BlockSpec((_BM, _D), lambda rb, e: (rb, 0)),
        out_shape=jax.ShapeDtypeStruct((_B, _D), jnp.float32),
        scratch_shapes=[
            pltpu.VMEM((_E, _BM, _D), jnp.float32),
            pltpu.VMEM((_E, 1, _BM), jnp.float32),
            pltpu.VMEM((_E, _BM), jnp.float32),
        ],
        compiler_params=pltpu.CompilerParams(
            dimension_semantics=("parallel", "arbitrary")),
    )
    return f(x, Wg, bg.reshape(_E, 1), We, be.reshape(_E, 1, _D))
